# Initial kernel scaffold; baseline (speedup 1.0000x reference)
import jax
import jax.numpy as jnp
from jax.experimental import pallas as pl


def kernel(x, edge_index, edge_attr, batch, w_dn_node, b_dn_node, w_dn_edge, b_dn_edge, bn1_node_w, bn1_node_b, bn1_edge_w, bn1_edge_b, conv_w, w_up_node, b_up_node, w_up_edge, b_up_edge, bn2_node_w, bn2_node_b, bn2_edge_w, bn2_edge_b):
    raise NotImplementedError("write your pallas kernel here")


# speedup vs baseline: 8.3200x; 8.3200x over previous
import functools

import numpy as np
import jax
import jax.numpy as jnp
from jax import lax
from jax.experimental import pallas as pl
from jax.experimental.pallas import tpu as pltpu
from jax.experimental.pallas import tpu_sc as plsc

N = 10000
E = 160000
C = 256
H = C // 4
ALPHA = 0.3678
EPS = 1e-5

NTILES = 32
NPAD = 10240
ROWS_T = NPAD // 16
EPT = 5120
EPAD = NTILES * EPT
CH = 128
NCH = EPT // CH

RB_E = 4000
RB_N = 2000

_PAD_SRC = np.arange(EPAD - E, dtype=np.int32) % N
_PAD_DST = N + np.arange(EPAD - E, dtype=np.int32) % (NPAD - N)


def _silu(z):
    return z * (1.0 / (1.0 + jnp.exp(-z)))


def _mean_inv(st_ref, cnt):
    m = st_ref[0:1, :] / cnt
    var = st_ref[1:2, :] / cnt - m * m
    return m, lax.rsqrt(var + EPS)


def _proj_stats_body(x_ref, w_ref, b_ref, o_ref, st_ref, acc_ref):
    i = pl.program_id(0)
    nb = pl.num_programs(0)
    pre = jnp.dot(x_ref[...], w_ref[...], preferred_element_type=jnp.float32)
    pre = pre + b_ref[...]
    o_ref[...] = pre

    @pl.when(i == 0)
    def _():
        acc_ref[...] = jnp.zeros_like(acc_ref)

    acc_ref[0:1, :] += jnp.sum(pre, axis=0, keepdims=True)
    acc_ref[1:2, :] += jnp.sum(pre * pre, axis=0, keepdims=True)

    @pl.when(i == nb - 1)
    def _():
        st_ref[...] = acc_ref[...]


def _proj_stats(x, w, b, rows, rb):
    din, dout = w.shape
    return pl.pallas_call(
        _proj_stats_body,
        grid=(rows // rb,),
        in_specs=[
            pl.BlockSpec((rb, din), lambda i: (i, 0)),
            pl.BlockSpec((din, dout), lambda i: (0, 0)),
            pl.BlockSpec((1, dout), lambda i: (0, 0)),
        ],
        out_specs=[
            pl.BlockSpec((rb, dout), lambda i: (i, 0)),
            pl.BlockSpec((2, dout), lambda i: (0, 0)),
        ],
        out_shape=[
            jax.ShapeDtypeStruct((rows, dout), jnp.float32),
            jax.ShapeDtypeStruct((2, dout), jnp.float32),
        ],
        scratch_shapes=[pltpu.VMEM((2, dout), jnp.float32)],
    )(x, w.reshape(din, dout), b.reshape(1, dout))


def _bn_silu_proj_body(cnt, pre_ref, st1_ref, g_ref, bb_ref, w_ref, b2_ref,
                       h_ref, st2_ref, acc_ref):
    i = pl.program_id(0)
    nb = pl.num_programs(0)
    m, inv = _mean_inv(st1_ref, cnt)
    z = (pre_ref[...] - m) * inv * g_ref[...] + bb_ref[...]
    h = _silu(z)
    h_ref[...] = h
    u = jnp.dot(h, w_ref[...], preferred_element_type=jnp.float32) + b2_ref[...]

    @pl.when(i == 0)
    def _():
        acc_ref[...] = jnp.zeros_like(acc_ref)

    acc_ref[0:1, :] += jnp.sum(u, axis=0, keepdims=True)
    acc_ref[1:2, :] += jnp.sum(u * u, axis=0, keepdims=True)

    @pl.when(i == nb - 1)
    def _():
        st2_ref[...] = acc_ref[...]


def _bn_silu_proj(pre, st1, g, bb, w, b2, rows, rb):
    din, dout = w.shape
    return pl.pallas_call(
        functools.partial(_bn_silu_proj_body, float(rows)),
        grid=(rows // rb,),
        in_specs=[
            pl.BlockSpec((rb, din), lambda i: (i, 0)),
            pl.BlockSpec((2, din), lambda i: (0, 0)),
            pl.BlockSpec((1, din), lambda i: (0, 0)),
            pl.BlockSpec((1, din), lambda i: (0, 0)),
            pl.BlockSpec((din, dout), lambda i: (0, 0)),
            pl.BlockSpec((1, dout), lambda i: (0, 0)),
        ],
        out_specs=[
            pl.BlockSpec((rb, din), lambda i: (i, 0)),
            pl.BlockSpec((2, dout), lambda i: (0, 0)),
        ],
        out_shape=[
            jax.ShapeDtypeStruct((rows, din), jnp.float32),
            jax.ShapeDtypeStruct((2, dout), jnp.float32),
        ],
        scratch_shapes=[pltpu.VMEM((2, dout), jnp.float32)],
    )(pre, st1, g.reshape(1, din), bb.reshape(1, din), w, b2.reshape(1, dout))


def _edge_out_body(cnt, h_ref, st2_ref, g_ref, bb_ref, w_ref, b2_ref, res_ref,
                   o_ref):
    m, inv = _mean_inv(st2_ref, cnt)
    u = jnp.dot(h_ref[...], w_ref[...], preferred_element_type=jnp.float32)
    u = u + b2_ref[...]
    z = (u - m) * inv * g_ref[...] + bb_ref[...] + res_ref[...]
    o_ref[...] = _silu(z)


def _edge_out(h, st2, g, bb, w, b2, res, rows, rb):
    din, dout = w.shape
    return pl.pallas_call(
        functools.partial(_edge_out_body, float(rows)),
        grid=(rows // rb,),
        in_specs=[
            pl.BlockSpec((rb, din), lambda i: (i, 0)),
            pl.BlockSpec((2, dout), lambda i: (0, 0)),
            pl.BlockSpec((1, dout), lambda i: (0, 0)),
            pl.BlockSpec((1, dout), lambda i: (0, 0)),
            pl.BlockSpec((din, dout), lambda i: (0, 0)),
            pl.BlockSpec((1, dout), lambda i: (0, 0)),
            pl.BlockSpec((rb, dout), lambda i: (i, 0)),
        ],
        out_specs=pl.BlockSpec((rb, dout), lambda i: (i, 0)),
        out_shape=jax.ShapeDtypeStruct((rows, dout), jnp.float32),
    )(h, st2, g.reshape(1, dout), bb.reshape(1, dout), w, b2.reshape(1, dout),
      res)


def _node_mid_body(cnt, pre_ref, st1_ref, g_ref, bb_ref, d0_ref, d1_ref,
                   h_ref, hp_ref, db_ref):
    m, inv = _mean_inv(st1_ref, cnt)
    z = (pre_ref[...] - m) * inv * g_ref[...] + bb_ref[...]
    h = _silu(z)
    deg = d0_ref[...] + d1_ref[...] + 1.0
    dis = lax.rsqrt(deg)
    h_ref[...] = h
    hp_ref[...] = h * dis
    db_ref[...] = jnp.broadcast_to(dis, h.shape)


def _node_mid(pre, st1, g, bb, d0, d1):
    return pl.pallas_call(
        functools.partial(_node_mid_body, float(N)),
        grid=(N // RB_N,),
        in_specs=[
            pl.BlockSpec((RB_N, H), lambda i: (i, 0)),
            pl.BlockSpec((2, H), lambda i: (0, 0)),
            pl.BlockSpec((1, H), lambda i: (0, 0)),
            pl.BlockSpec((1, H), lambda i: (0, 0)),
            pl.BlockSpec((RB_N, 1), lambda i: (i, 0)),
            pl.BlockSpec((RB_N, 1), lambda i: (i, 0)),
        ],
        out_specs=[
            pl.BlockSpec((RB_N, H), lambda i: (i, 0)),
            pl.BlockSpec((RB_N, H), lambda i: (i, 0)),
            pl.BlockSpec((RB_N, H), lambda i: (i, 0)),
        ],
        out_shape=[
            jax.ShapeDtypeStruct((N, H), jnp.float32),
            jax.ShapeDtypeStruct((N, H), jnp.float32),
            jax.ShapeDtypeStruct((N, H), jnp.float32),
        ],
    )(pre, st1, g.reshape(1, H), bb.reshape(1, H), d0, d1)


def _node_conv_body(cnt, p0_ref, p1_ref, h_ref, db_ref, cw_ref, wu_ref,
                    bu_ref, u_ref, st_ref, acc_ref):
    i = pl.program_id(0)
    nb = pl.num_programs(0)
    db = db_ref[...]
    h = h_ref[...]
    agg = db * (p0_ref[...] + p1_ref[...]) + db * db * h
    support = (1.0 - ALPHA) * agg + ALPHA * h
    conv = jnp.dot(support, cw_ref[...], preferred_element_type=jnp.float32)
    u = jnp.dot(conv, wu_ref[...], preferred_element_type=jnp.float32)
    u = u + bu_ref[...]
    u_ref[...] = u

    @pl.when(i == 0)
    def _():
        acc_ref[...] = jnp.zeros_like(acc_ref)

    acc_ref[0:1, :] += jnp.sum(u, axis=0, keepdims=True)
    acc_ref[1:2, :] += jnp.sum(u * u, axis=0, keepdims=True)

    @pl.when(i == nb - 1)
    def _():
        st_ref[...] = acc_ref[...]


def _node_conv(p0, p1, h, db, cw, wu, bu):
    return pl.pallas_call(
        functools.partial(_node_conv_body, float(N)),
        grid=(N // RB_N,),
        in_specs=[
            pl.BlockSpec((RB_N, H), lambda i: (i, 0)),
            pl.BlockSpec((RB_N, H), lambda i: (i, 0)),
            pl.BlockSpec((RB_N, H), lambda i: (i, 0)),
            pl.BlockSpec((RB_N, H), lambda i: (i, 0)),
            pl.BlockSpec((H, H), lambda i: (0, 0)),
            pl.BlockSpec((H, C), lambda i: (0, 0)),
            pl.BlockSpec((1, C), lambda i: (0, 0)),
        ],
        out_specs=[
            pl.BlockSpec((RB_N, C), lambda i: (i, 0)),
            pl.BlockSpec((2, C), lambda i: (0, 0)),
        ],
        out_shape=[
            jax.ShapeDtypeStruct((N, C), jnp.float32),
            jax.ShapeDtypeStruct((2, C), jnp.float32),
        ],
        scratch_shapes=[pltpu.VMEM((2, C), jnp.float32)],
    )(p0, p1, h, db, cw, wu, bu.reshape(1, C))


def _node_out_body(cnt, u_ref, st_ref, g_ref, bb_ref, x_ref, o_ref):
    m, inv = _mean_inv(st_ref, cnt)
    z = (u_ref[...] - m) * inv * g_ref[...] + bb_ref[...] + x_ref[...]
    o_ref[...] = _silu(z)


def _node_out(u, st, g, bb, x):
    return pl.pallas_call(
        functools.partial(_node_out_body, float(N)),
        grid=(N // RB_N,),
        in_specs=[
            pl.BlockSpec((RB_N, C), lambda i: (i, 0)),
            pl.BlockSpec((2, C), lambda i: (0, 0)),
            pl.BlockSpec((1, C), lambda i: (0, 0)),
            pl.BlockSpec((1, C), lambda i: (0, 0)),
            pl.BlockSpec((RB_N, C), lambda i: (i, 0)),
        ],
        out_specs=pl.BlockSpec((RB_N, C), lambda i: (i, 0)),
        out_shape=jax.ShapeDtypeStruct((N, C), jnp.float32),
    )(u, st, g.reshape(1, C), bb.reshape(1, C), x)


def _sc_deg_body(dst_hbm, out_hbm, idx_v, ones_v, zrow_v, acc_sh):
    c = lax.axis_index("c")
    s = lax.axis_index("s")
    wid = c * 16 + s
    base = wid * EPT
    for j in range(CH // 16):
        ones_v[pl.ds(j * 16, 16)] = jnp.ones((16,), jnp.float32)
    for j in range(ROWS_T // 16):
        zrow_v[pl.ds(j * 16, 16)] = jnp.zeros((16,), jnp.float32)
    pltpu.sync_copy(zrow_v, acc_sh.at[pl.ds(s * ROWS_T, ROWS_T)])
    plsc.subcore_barrier()

    def body(i, carry):
        pltpu.sync_copy(dst_hbm.at[pl.ds(base + i * CH, CH)], idx_v)
        pltpu.sync_copy(ones_v, acc_sh.at[idx_v], add=True)
        return carry

    lax.fori_loop(0, NCH, body, 0)
    plsc.subcore_barrier()
    pltpu.sync_copy(acc_sh.at[pl.ds(s * ROWS_T, ROWS_T)],
                    out_hbm.at[c, pl.ds(s * ROWS_T, ROWS_T)])


@functools.cache
def _sc_deg():
    mesh = plsc.VectorSubcoreMesh(core_axis_name="c", subcore_axis_name="s")
    return pl.kernel(
        _sc_deg_body,
        out_type=jax.ShapeDtypeStruct((2, NPAD), jnp.float32),
        mesh=mesh,
        scratch_types=[
            pltpu.VMEM((CH,), jnp.int32),
            pltpu.VMEM((CH,), jnp.float32),
            pltpu.VMEM((ROWS_T,), jnp.float32),
            pltpu.VMEM_SHARED((NPAD,), jnp.float32),
        ],
    )


def _sc_agg_body(src_hbm, dst_hbm, hp_hbm, zeros_hbm, out_hbm,
                 isrc_v, idst_v, rows_v, acc_sh, sem):
    c = lax.axis_index("c")
    s = lax.axis_index("s")
    wid = c * 16 + s
    base = wid * EPT
    pltpu.sync_copy(zeros_hbm.at[pl.ds(s * ROWS_T, ROWS_T)],
                    acc_sh.at[pl.ds(s * ROWS_T, ROWS_T)])
    plsc.subcore_barrier()

    def body(i, carry):
        pltpu.sync_copy(src_hbm.at[pl.ds(base + i * CH, CH)], isrc_v)
        pltpu.sync_copy(dst_hbm.at[pl.ds(base + i * CH, CH)], idst_v)
        pltpu.async_copy(hp_hbm.at[isrc_v], rows_v, sem).wait()
        pltpu.sync_copy(rows_v, acc_sh.at[idst_v], add=True)
        return carry

    lax.fori_loop(0, NCH, body, 0)
    plsc.subcore_barrier()
    pltpu.sync_copy(acc_sh.at[pl.ds(s * ROWS_T, ROWS_T)],
                    out_hbm.at[c, pl.ds(s * ROWS_T, ROWS_T)])


@functools.cache
def _sc_agg():
    mesh = plsc.VectorSubcoreMesh(core_axis_name="c", subcore_axis_name="s")
    return pl.kernel(
        _sc_agg_body,
        out_type=jax.ShapeDtypeStruct((2, NPAD, H), jnp.float32),
        mesh=mesh,
        scratch_types=[
            pltpu.VMEM((CH,), jnp.int32),
            pltpu.VMEM((CH,), jnp.int32),
            pltpu.VMEM((CH, H), jnp.float32),
            pltpu.VMEM_SHARED((NPAD, H), jnp.float32),
            pltpu.SemaphoreType.DMA,
        ],
        compiler_params=pltpu.CompilerParams(use_tc_tiling_on_sc=False),
    )


def _sc_deg_call(dst):
    return _sc_deg()(dst)


def _sc_agg_call(src, dst, hp, zeros):
    return _sc_agg()(src, dst, hp, zeros)


def kernel(x, edge_index, edge_attr, batch,
           w_dn_node, b_dn_node, w_dn_edge, b_dn_edge,
           bn1_node_w, bn1_node_b, bn1_edge_w, bn1_edge_b,
           conv_w,
           w_up_node, b_up_node, w_up_edge, b_up_edge,
           bn2_node_w, bn2_node_b, bn2_edge_w, bn2_edge_b):
    src = jnp.concatenate([edge_index[0], jnp.asarray(_PAD_SRC)])
    dst = jnp.concatenate([edge_index[1], jnp.asarray(_PAD_DST)])

    deg_p = _sc_deg_call(dst)
    d0 = deg_p[0].reshape(NPAD, 1)
    d1 = deg_p[1].reshape(NPAD, 1)

    pre_n, st1n = _proj_stats(x, w_dn_node, b_dn_node, N, RB_N)
    pre_e, st1e = _proj_stats(edge_attr, w_dn_edge, b_dn_edge, E, RB_E)

    h_node, hp, db = _node_mid(pre_n, st1n, bn1_node_w, bn1_node_b, d0, d1)

    zeros = jnp.zeros((NPAD, H), jnp.float32)
    agg_p = _sc_agg_call(src, dst, hp, zeros)

    u_n, st2n = _node_conv(agg_p[0], agg_p[1], h_node, db,
                           conv_w, w_up_node, b_up_node)
    x_out = _node_out(u_n, st2n, bn2_node_w, bn2_node_b, x)

    h_edge, st2e = _bn_silu_proj(pre_e, st1e, bn1_edge_w, bn1_edge_b,
                                 w_up_edge, b_up_edge, E, RB_E)
    edge_out = _edge_out(h_edge, st2e, bn2_edge_w, bn2_edge_b,
                         w_up_edge, b_up_edge, edge_attr, E, RB_E)
    return x_out, edge_out


# speedup vs baseline: 8.5032x; 1.0220x over previous
import functools

import numpy as np
import jax
import jax.numpy as jnp
from jax import lax
from jax.experimental import pallas as pl
from jax.experimental.pallas import tpu as pltpu
from jax.experimental.pallas import tpu_sc as plsc

N = 10000
E = 160000
C = 256
H = C // 4
ALPHA = 0.3678
EPS = 1e-5

NTILES = 32
NPAD = 10240
ROWS_T = NPAD // 16
EPT = 5120
EPAD = NTILES * EPT
CH = 128
NCH = EPT // CH

RB_E = 4000
RB_N = 2000

_PAD_SRC = np.arange(EPAD - E, dtype=np.int32) % N
_PAD_DST = N + np.arange(EPAD - E, dtype=np.int32) % (NPAD - N)


def _silu(z):
    return z * (1.0 / (1.0 + jnp.exp(-z)))


def _mean_inv(st_ref, cnt):
    m = st_ref[0:1, :] / cnt
    var = st_ref[1:2, :] / cnt - m * m
    return m, lax.rsqrt(var + EPS)


def _proj_stats_body(x_ref, w_ref, b_ref, o_ref, st_ref, acc_ref):
    i = pl.program_id(0)
    nb = pl.num_programs(0)
    pre = jnp.dot(x_ref[...], w_ref[...], preferred_element_type=jnp.float32)
    pre = pre + b_ref[...]
    o_ref[...] = pre

    @pl.when(i == 0)
    def _():
        acc_ref[...] = jnp.zeros_like(acc_ref)

    acc_ref[0:1, :] += jnp.sum(pre, axis=0, keepdims=True)
    acc_ref[1:2, :] += jnp.sum(pre * pre, axis=0, keepdims=True)

    @pl.when(i == nb - 1)
    def _():
        st_ref[...] = acc_ref[...]


def _proj_stats(x, w, b, rows, rb):
    din, dout = w.shape
    return pl.pallas_call(
        _proj_stats_body,
        grid=(rows // rb,),
        in_specs=[
            pl.BlockSpec((rb, din), lambda i: (i, 0)),
            pl.BlockSpec((din, dout), lambda i: (0, 0)),
            pl.BlockSpec((1, dout), lambda i: (0, 0)),
        ],
        out_specs=[
            pl.BlockSpec((rb, dout), lambda i: (i, 0)),
            pl.BlockSpec((2, dout), lambda i: (0, 0)),
        ],
        out_shape=[
            jax.ShapeDtypeStruct((rows, dout), jnp.float32),
            jax.ShapeDtypeStruct((2, dout), jnp.float32),
        ],
        scratch_shapes=[pltpu.VMEM((2, dout), jnp.float32)],
    )(x, w.reshape(din, dout), b.reshape(1, dout))


def _bn_silu_proj_body(cnt, pre_ref, st1_ref, g_ref, bb_ref, w_ref, b2_ref,
                       h_ref, st2_ref, acc_ref):
    i = pl.program_id(0)
    nb = pl.num_programs(0)
    m, inv = _mean_inv(st1_ref, cnt)
    z = (pre_ref[...] - m) * inv * g_ref[...] + bb_ref[...]
    h = _silu(z)
    h_ref[...] = h
    u = jnp.dot(h, w_ref[...], preferred_element_type=jnp.float32) + b2_ref[...]

    @pl.when(i == 0)
    def _():
        acc_ref[...] = jnp.zeros_like(acc_ref)

    acc_ref[0:1, :] += jnp.sum(u, axis=0, keepdims=True)
    acc_ref[1:2, :] += jnp.sum(u * u, axis=0, keepdims=True)

    @pl.when(i == nb - 1)
    def _():
        st2_ref[...] = acc_ref[...]


def _bn_silu_proj(pre, st1, g, bb, w, b2, rows, rb):
    din, dout = w.shape
    return pl.pallas_call(
        functools.partial(_bn_silu_proj_body, float(rows)),
        grid=(rows // rb,),
        in_specs=[
            pl.BlockSpec((rb, din), lambda i: (i, 0)),
            pl.BlockSpec((2, din), lambda i: (0, 0)),
            pl.BlockSpec((1, din), lambda i: (0, 0)),
            pl.BlockSpec((1, din), lambda i: (0, 0)),
            pl.BlockSpec((din, dout), lambda i: (0, 0)),
            pl.BlockSpec((1, dout), lambda i: (0, 0)),
        ],
        out_specs=[
            pl.BlockSpec((rb, din), lambda i: (i, 0)),
            pl.BlockSpec((2, dout), lambda i: (0, 0)),
        ],
        out_shape=[
            jax.ShapeDtypeStruct((rows, din), jnp.float32),
            jax.ShapeDtypeStruct((2, dout), jnp.float32),
        ],
        scratch_shapes=[pltpu.VMEM((2, dout), jnp.float32)],
    )(pre, st1, g.reshape(1, din), bb.reshape(1, din), w, b2.reshape(1, dout))


def _edge_out_body(cnt, h_ref, st2_ref, g_ref, bb_ref, w_ref, b2_ref, res_ref,
                   o_ref):
    m, inv = _mean_inv(st2_ref, cnt)
    u = jnp.dot(h_ref[...], w_ref[...], preferred_element_type=jnp.float32)
    u = u + b2_ref[...]
    z = (u - m) * inv * g_ref[...] + bb_ref[...] + res_ref[...]
    o_ref[...] = _silu(z)


def _edge_out(h, st2, g, bb, w, b2, res, rows, rb):
    din, dout = w.shape
    return pl.pallas_call(
        functools.partial(_edge_out_body, float(rows)),
        grid=(rows // rb,),
        in_specs=[
            pl.BlockSpec((rb, din), lambda i: (i, 0)),
            pl.BlockSpec((2, dout), lambda i: (0, 0)),
            pl.BlockSpec((1, dout), lambda i: (0, 0)),
            pl.BlockSpec((1, dout), lambda i: (0, 0)),
            pl.BlockSpec((din, dout), lambda i: (0, 0)),
            pl.BlockSpec((1, dout), lambda i: (0, 0)),
            pl.BlockSpec((rb, dout), lambda i: (i, 0)),
        ],
        out_specs=pl.BlockSpec((rb, dout), lambda i: (i, 0)),
        out_shape=jax.ShapeDtypeStruct((rows, dout), jnp.float32),
    )(h, st2, g.reshape(1, dout), bb.reshape(1, dout), w, b2.reshape(1, dout),
      res)


def _node_mid_body(cnt, pre_ref, st1_ref, g_ref, bb_ref, d0_ref, d1_ref,
                   h_ref, hp_ref, db_ref):
    m, inv = _mean_inv(st1_ref, cnt)
    z = (pre_ref[...] - m) * inv * g_ref[...] + bb_ref[...]
    h = _silu(z)
    deg = d0_ref[...] + d1_ref[...] + 1.0
    dis = lax.rsqrt(deg)
    h_ref[...] = h
    hp_ref[...] = h * dis
    db_ref[...] = jnp.broadcast_to(dis, h.shape)


def _node_mid(pre, st1, g, bb, d0, d1):
    return pl.pallas_call(
        functools.partial(_node_mid_body, float(N)),
        grid=(N // RB_N,),
        in_specs=[
            pl.BlockSpec((RB_N, H), lambda i: (i, 0)),
            pl.BlockSpec((2, H), lambda i: (0, 0)),
            pl.BlockSpec((1, H), lambda i: (0, 0)),
            pl.BlockSpec((1, H), lambda i: (0, 0)),
            pl.BlockSpec((RB_N, 1), lambda i: (i, 0)),
            pl.BlockSpec((RB_N, 1), lambda i: (i, 0)),
        ],
        out_specs=[
            pl.BlockSpec((RB_N, H), lambda i: (i, 0)),
            pl.BlockSpec((RB_N, H), lambda i: (i, 0)),
            pl.BlockSpec((RB_N, H), lambda i: (i, 0)),
        ],
        out_shape=[
            jax.ShapeDtypeStruct((N, H), jnp.float32),
            jax.ShapeDtypeStruct((N, H), jnp.float32),
            jax.ShapeDtypeStruct((N, H), jnp.float32),
        ],
    )(pre, st1, g.reshape(1, H), bb.reshape(1, H), d0, d1)


def _node_conv_body(cnt, p0_ref, p1_ref, h_ref, db_ref, cw_ref, wu_ref,
                    bu_ref, u_ref, st_ref, acc_ref):
    i = pl.program_id(0)
    nb = pl.num_programs(0)
    db = db_ref[...]
    h = h_ref[...]
    agg = db * (p0_ref[...] + p1_ref[...]) + db * db * h
    support = (1.0 - ALPHA) * agg + ALPHA * h
    conv = jnp.dot(support, cw_ref[...], preferred_element_type=jnp.float32)
    u = jnp.dot(conv, wu_ref[...], preferred_element_type=jnp.float32)
    u = u + bu_ref[...]
    u_ref[...] = u

    @pl.when(i == 0)
    def _():
        acc_ref[...] = jnp.zeros_like(acc_ref)

    acc_ref[0:1, :] += jnp.sum(u, axis=0, keepdims=True)
    acc_ref[1:2, :] += jnp.sum(u * u, axis=0, keepdims=True)

    @pl.when(i == nb - 1)
    def _():
        st_ref[...] = acc_ref[...]


def _node_conv(p0, p1, h, db, cw, wu, bu):
    return pl.pallas_call(
        functools.partial(_node_conv_body, float(N)),
        grid=(N // RB_N,),
        in_specs=[
            pl.BlockSpec((RB_N, H), lambda i: (i, 0)),
            pl.BlockSpec((RB_N, H), lambda i: (i, 0)),
            pl.BlockSpec((RB_N, H), lambda i: (i, 0)),
            pl.BlockSpec((RB_N, H), lambda i: (i, 0)),
            pl.BlockSpec((H, H), lambda i: (0, 0)),
            pl.BlockSpec((H, C), lambda i: (0, 0)),
            pl.BlockSpec((1, C), lambda i: (0, 0)),
        ],
        out_specs=[
            pl.BlockSpec((RB_N, C), lambda i: (i, 0)),
            pl.BlockSpec((2, C), lambda i: (0, 0)),
        ],
        out_shape=[
            jax.ShapeDtypeStruct((N, C), jnp.float32),
            jax.ShapeDtypeStruct((2, C), jnp.float32),
        ],
        scratch_shapes=[pltpu.VMEM((2, C), jnp.float32)],
    )(p0, p1, h, db, cw, wu, bu.reshape(1, C))


def _node_out_body(cnt, u_ref, st_ref, g_ref, bb_ref, x_ref, o_ref):
    m, inv = _mean_inv(st_ref, cnt)
    z = (u_ref[...] - m) * inv * g_ref[...] + bb_ref[...] + x_ref[...]
    o_ref[...] = _silu(z)


def _node_out(u, st, g, bb, x):
    return pl.pallas_call(
        functools.partial(_node_out_body, float(N)),
        grid=(N // RB_N,),
        in_specs=[
            pl.BlockSpec((RB_N, C), lambda i: (i, 0)),
            pl.BlockSpec((2, C), lambda i: (0, 0)),
            pl.BlockSpec((1, C), lambda i: (0, 0)),
            pl.BlockSpec((1, C), lambda i: (0, 0)),
            pl.BlockSpec((RB_N, C), lambda i: (i, 0)),
        ],
        out_specs=pl.BlockSpec((RB_N, C), lambda i: (i, 0)),
        out_shape=jax.ShapeDtypeStruct((N, C), jnp.float32),
    )(u, st, g.reshape(1, C), bb.reshape(1, C), x)


def _sc_deg_body(idx_hbm, out_hbm, slab_v, ones_v, zrow_v, acc_sh):
    c = lax.axis_index("c")
    s = lax.axis_index("s")
    wid = c * 16 + s
    pltpu.sync_copy(idx_hbm.at[pl.ds(wid * 2 * NCH, 2 * NCH)], slab_v)
    for j in range(CH // 16):
        ones_v[pl.ds(j * 16, 16)] = jnp.ones((16,), jnp.float32)
    for j in range(ROWS_T // 16):
        zrow_v[pl.ds(j * 16, 16)] = jnp.zeros((16,), jnp.float32)
    pltpu.sync_copy(zrow_v, acc_sh.at[pl.ds(s * ROWS_T, ROWS_T)])
    plsc.subcore_barrier()

    def body(i, carry):
        pltpu.sync_copy(ones_v, acc_sh.at[slab_v.at[2 * i + 1]], add=True)
        return carry

    lax.fori_loop(0, NCH, body, 0)
    plsc.subcore_barrier()
    pltpu.sync_copy(acc_sh.at[pl.ds(s * ROWS_T, ROWS_T)],
                    out_hbm.at[c, pl.ds(s * ROWS_T, ROWS_T)])


@functools.cache
def _sc_deg():
    mesh = plsc.VectorSubcoreMesh(core_axis_name="c", subcore_axis_name="s")
    return pl.kernel(
        _sc_deg_body,
        out_type=jax.ShapeDtypeStruct((2, NPAD), jnp.float32),
        mesh=mesh,
        scratch_types=[
            pltpu.VMEM((2 * NCH, CH), jnp.int32),
            pltpu.VMEM((CH,), jnp.float32),
            pltpu.VMEM((ROWS_T,), jnp.float32),
            pltpu.VMEM_SHARED((NPAD,), jnp.float32),
        ],
        compiler_params=pltpu.CompilerParams(use_tc_tiling_on_sc=False),
    )


def _sc_agg_body(idx_hbm, hp_hbm, zeros_hbm, out_hbm,
                 slab_v, rows0_v, rows1_v, acc_sh, sem0, sem1):
    c = lax.axis_index("c")
    s = lax.axis_index("s")
    wid = c * 16 + s
    rows = (rows0_v, rows1_v)
    sems = (sem0, sem1)
    pltpu.sync_copy(idx_hbm.at[pl.ds(wid * 2 * NCH, 2 * NCH)], slab_v)
    pltpu.sync_copy(zeros_hbm.at[pl.ds(s * ROWS_T, ROWS_T)],
                    acc_sh.at[pl.ds(s * ROWS_T, ROWS_T)])
    plsc.subcore_barrier()

    for p in range(2):
        pltpu.async_copy(hp_hbm.at[slab_v.at[2 * p]], rows[p], sems[p])

    def body(k, carry):
        for p in range(2):
            i = 2 * k + p
            pltpu.make_async_copy(hp_hbm.at[slab_v.at[0]], rows[p],
                                  sems[p]).wait()
            pltpu.sync_copy(rows[p], acc_sh.at[slab_v.at[2 * i + 1]],
                            add=True)
            pltpu.async_copy(hp_hbm.at[slab_v.at[2 * (i + 2)]], rows[p],
                             sems[p])
        return carry

    lax.fori_loop(0, NCH // 2 - 1, body, 0)
    for p in range(2):
        i = NCH - 2 + p
        pltpu.make_async_copy(hp_hbm.at[slab_v.at[0]], rows[p],
                              sems[p]).wait()
        pltpu.sync_copy(rows[p], acc_sh.at[slab_v.at[2 * i + 1]], add=True)
    plsc.subcore_barrier()
    pltpu.sync_copy(acc_sh.at[pl.ds(s * ROWS_T, ROWS_T)],
                    out_hbm.at[c, pl.ds(s * ROWS_T, ROWS_T)])


@functools.cache
def _sc_agg():
    mesh = plsc.VectorSubcoreMesh(core_axis_name="c", subcore_axis_name="s")
    return pl.kernel(
        _sc_agg_body,
        out_type=jax.ShapeDtypeStruct((2, NPAD, H), jnp.float32),
        mesh=mesh,
        scratch_types=[
            pltpu.VMEM((2 * NCH, CH), jnp.int32),
            pltpu.VMEM((CH, H), jnp.float32),
            pltpu.VMEM((CH, H), jnp.float32),
            pltpu.VMEM_SHARED((NPAD, H), jnp.float32),
            pltpu.SemaphoreType.DMA,
            pltpu.SemaphoreType.DMA,
        ],
        compiler_params=pltpu.CompilerParams(use_tc_tiling_on_sc=False),
    )


def _interleave_idx(src, dst):
    s2 = src.reshape(NTILES * NCH, CH)
    d2 = dst.reshape(NTILES * NCH, CH)
    return jnp.stack([s2, d2], axis=1).reshape(NTILES * 2 * NCH, CH)


def _sc_deg_call(idx_il):
    return _sc_deg()(idx_il)


def _sc_agg_call(idx_il, hp, zeros):
    return _sc_agg()(idx_il, hp, zeros)


def kernel(x, edge_index, edge_attr, batch,
           w_dn_node, b_dn_node, w_dn_edge, b_dn_edge,
           bn1_node_w, bn1_node_b, bn1_edge_w, bn1_edge_b,
           conv_w,
           w_up_node, b_up_node, w_up_edge, b_up_edge,
           bn2_node_w, bn2_node_b, bn2_edge_w, bn2_edge_b):
    src = jnp.concatenate([edge_index[0], jnp.asarray(_PAD_SRC)])
    dst = jnp.concatenate([edge_index[1], jnp.asarray(_PAD_DST)])
    idx_il = _interleave_idx(src, dst)

    deg_p = _sc_deg_call(idx_il)
    d0 = deg_p[0].reshape(NPAD, 1)
    d1 = deg_p[1].reshape(NPAD, 1)

    pre_n, st1n = _proj_stats(x, w_dn_node, b_dn_node, N, RB_N)
    pre_e, st1e = _proj_stats(edge_attr, w_dn_edge, b_dn_edge, E, RB_E)

    h_node, hp, db = _node_mid(pre_n, st1n, bn1_node_w, bn1_node_b, d0, d1)

    zeros = jnp.zeros((NPAD, H), jnp.float32)
    agg_p = _sc_agg_call(idx_il, hp, zeros)

    u_n, st2n = _node_conv(agg_p[0], agg_p[1], h_node, db,
                           conv_w, w_up_node, b_up_node)
    x_out = _node_out(u_n, st2n, bn2_node_w, bn2_node_b, x)

    h_edge, st2e = _bn_silu_proj(pre_e, st1e, bn1_edge_w, bn1_edge_b,
                                 w_up_edge, b_up_edge, E, RB_E)
    edge_out = _edge_out(h_edge, st2e, bn2_edge_w, bn2_edge_b,
                         w_up_edge, b_up_edge, edge_attr, E, RB_E)
    return x_out, edge_out


# speedup vs baseline: 9.1867x; 1.0804x over previous
import functools

import numpy as np
import jax
import jax.numpy as jnp
from jax import lax
from jax.experimental import pallas as pl
from jax.experimental.pallas import tpu as pltpu
from jax.experimental.pallas import tpu_sc as plsc

N = 10000
E = 160000
C = 256
H = C // 4
ALPHA = 0.3678
EPS = 1e-5

NTILES = 32
NPAD = 10240
ROWS_T = NPAD // 16
EPT = 5120
EPAD = NTILES * EPT
CH = 128
NCH = EPT // CH

RB_E = 4000
RB_N = 2000

_PAD_SRC = np.arange(EPAD - E, dtype=np.int32) % N
_PAD_DST = N + np.arange(EPAD - E, dtype=np.int32) % (NPAD - N)


def _silu(z):
    return z * (1.0 / (1.0 + jnp.exp(-z)))


def _mean_inv(st_ref, cnt):
    m = st_ref[0:1, :] / cnt
    var = st_ref[1:2, :] / cnt - m * m
    return m, lax.rsqrt(var + EPS)


def _proj_stats_body(x_ref, w_ref, b_ref, o_ref, st_ref, acc_ref):
    i = pl.program_id(0)
    nb = pl.num_programs(0)
    pre = jnp.dot(x_ref[...], w_ref[...], preferred_element_type=jnp.float32)
    pre = pre + b_ref[...]
    o_ref[...] = pre.astype(o_ref.dtype)

    @pl.when(i == 0)
    def _():
        acc_ref[...] = jnp.zeros_like(acc_ref)

    acc_ref[0:1, :] += jnp.sum(pre, axis=0, keepdims=True)
    acc_ref[1:2, :] += jnp.sum(pre * pre, axis=0, keepdims=True)

    @pl.when(i == nb - 1)
    def _():
        st_ref[...] = acc_ref[...]


def _proj_stats(x, w, b, rows, rb, out_dtype=jnp.float32):
    din, dout = w.shape
    return pl.pallas_call(
        _proj_stats_body,
        grid=(rows // rb,),
        in_specs=[
            pl.BlockSpec((rb, din), lambda i: (i, 0)),
            pl.BlockSpec((din, dout), lambda i: (0, 0)),
            pl.BlockSpec((1, dout), lambda i: (0, 0)),
        ],
        out_specs=[
            pl.BlockSpec((rb, dout), lambda i: (i, 0)),
            pl.BlockSpec((2, dout), lambda i: (0, 0)),
        ],
        out_shape=[
            jax.ShapeDtypeStruct((rows, dout), out_dtype),
            jax.ShapeDtypeStruct((2, dout), jnp.float32),
        ],
        scratch_shapes=[pltpu.VMEM((2, dout), jnp.float32)],
    )(x, w.reshape(din, dout), b.reshape(1, dout))


def _bn_silu_proj_body(cnt, pre_ref, st1_ref, g_ref, bb_ref, w_ref, b2_ref,
                       h_ref, st2_ref, acc_ref):
    i = pl.program_id(0)
    nb = pl.num_programs(0)
    m, inv = _mean_inv(st1_ref, cnt)
    z = (pre_ref[...].astype(jnp.float32) - m) * inv * g_ref[...] + bb_ref[...]
    h = _silu(z).astype(h_ref.dtype)
    h_ref[...] = h
    u = jnp.dot(h.astype(jnp.float32), w_ref[...],
                preferred_element_type=jnp.float32) + b2_ref[...]

    @pl.when(i == 0)
    def _():
        acc_ref[...] = jnp.zeros_like(acc_ref)

    acc_ref[0:1, :] += jnp.sum(u, axis=0, keepdims=True)
    acc_ref[1:2, :] += jnp.sum(u * u, axis=0, keepdims=True)

    @pl.when(i == nb - 1)
    def _():
        st2_ref[...] = acc_ref[...]


def _bn_silu_proj(pre, st1, g, bb, w, b2, rows, rb):
    din, dout = w.shape
    return pl.pallas_call(
        functools.partial(_bn_silu_proj_body, float(rows)),
        grid=(rows // rb,),
        in_specs=[
            pl.BlockSpec((rb, din), lambda i: (i, 0)),
            pl.BlockSpec((2, din), lambda i: (0, 0)),
            pl.BlockSpec((1, din), lambda i: (0, 0)),
            pl.BlockSpec((1, din), lambda i: (0, 0)),
            pl.BlockSpec((din, dout), lambda i: (0, 0)),
            pl.BlockSpec((1, dout), lambda i: (0, 0)),
        ],
        out_specs=[
            pl.BlockSpec((rb, din), lambda i: (i, 0)),
            pl.BlockSpec((2, dout), lambda i: (0, 0)),
        ],
        out_shape=[
            jax.ShapeDtypeStruct((rows, din), jnp.bfloat16),
            jax.ShapeDtypeStruct((2, dout), jnp.float32),
        ],
        scratch_shapes=[pltpu.VMEM((2, dout), jnp.float32)],
    )(pre, st1, g.reshape(1, din), bb.reshape(1, din), w, b2.reshape(1, dout))


def _edge_out_body(cnt, h_ref, st2_ref, g_ref, bb_ref, w_ref, b2_ref, res_ref,
                   o_ref):
    m, inv = _mean_inv(st2_ref, cnt)
    u = jnp.dot(h_ref[...].astype(jnp.float32), w_ref[...],
                preferred_element_type=jnp.float32)
    u = u + b2_ref[...]
    z = (u - m) * inv * g_ref[...] + bb_ref[...] + res_ref[...]
    o_ref[...] = _silu(z)


def _edge_out(h, st2, g, bb, w, b2, res, rows, rb):
    din, dout = w.shape
    return pl.pallas_call(
        functools.partial(_edge_out_body, float(rows)),
        grid=(rows // rb,),
        in_specs=[
            pl.BlockSpec((rb, din), lambda i: (i, 0)),
            pl.BlockSpec((2, dout), lambda i: (0, 0)),
            pl.BlockSpec((1, dout), lambda i: (0, 0)),
            pl.BlockSpec((1, dout), lambda i: (0, 0)),
            pl.BlockSpec((din, dout), lambda i: (0, 0)),
            pl.BlockSpec((1, dout), lambda i: (0, 0)),
            pl.BlockSpec((rb, dout), lambda i: (i, 0)),
        ],
        out_specs=pl.BlockSpec((rb, dout), lambda i: (i, 0)),
        out_shape=jax.ShapeDtypeStruct((rows, dout), jnp.float32),
    )(h, st2, g.reshape(1, dout), bb.reshape(1, dout), w, b2.reshape(1, dout),
      res)


def _node_mid_body(cnt, pre_ref, st1_ref, g_ref, bb_ref, d0_ref, d1_ref,
                   h_ref, hp_ref, db_ref):
    m, inv = _mean_inv(st1_ref, cnt)
    z = (pre_ref[...] - m) * inv * g_ref[...] + bb_ref[...]
    h = _silu(z)
    deg = d0_ref[...] + d1_ref[...] + 1.0
    dis = lax.rsqrt(deg)
    h_ref[...] = h
    hp_ref[...] = h * dis
    db_ref[...] = jnp.broadcast_to(dis, h.shape)


def _node_mid(pre, st1, g, bb, d0, d1):
    return pl.pallas_call(
        functools.partial(_node_mid_body, float(N)),
        grid=(N // RB_N,),
        in_specs=[
            pl.BlockSpec((RB_N, H), lambda i: (i, 0)),
            pl.BlockSpec((2, H), lambda i: (0, 0)),
            pl.BlockSpec((1, H), lambda i: (0, 0)),
            pl.BlockSpec((1, H), lambda i: (0, 0)),
            pl.BlockSpec((RB_N, 1), lambda i: (i, 0)),
            pl.BlockSpec((RB_N, 1), lambda i: (i, 0)),
        ],
        out_specs=[
            pl.BlockSpec((RB_N, H), lambda i: (i, 0)),
            pl.BlockSpec((RB_N, H), lambda i: (i, 0)),
            pl.BlockSpec((RB_N, H), lambda i: (i, 0)),
        ],
        out_shape=[
            jax.ShapeDtypeStruct((N, H), jnp.float32),
            jax.ShapeDtypeStruct((N, H), jnp.float32),
            jax.ShapeDtypeStruct((N, H), jnp.float32),
        ],
    )(pre, st1, g.reshape(1, H), bb.reshape(1, H), d0, d1)


def _node_conv_body(cnt, p0_ref, p1_ref, h_ref, db_ref, cw_ref, wu_ref,
                    bu_ref, u_ref, st_ref, acc_ref):
    i = pl.program_id(0)
    nb = pl.num_programs(0)
    db = db_ref[...]
    h = h_ref[...]
    agg = db * (p0_ref[...] + p1_ref[...]) + db * db * h
    support = (1.0 - ALPHA) * agg + ALPHA * h
    conv = jnp.dot(support, cw_ref[...], preferred_element_type=jnp.float32)
    u = jnp.dot(conv, wu_ref[...], preferred_element_type=jnp.float32)
    u = u + bu_ref[...]
    u_ref[...] = u

    @pl.when(i == 0)
    def _():
        acc_ref[...] = jnp.zeros_like(acc_ref)

    acc_ref[0:1, :] += jnp.sum(u, axis=0, keepdims=True)
    acc_ref[1:2, :] += jnp.sum(u * u, axis=0, keepdims=True)

    @pl.when(i == nb - 1)
    def _():
        st_ref[...] = acc_ref[...]


def _node_conv(p0, p1, h, db, cw, wu, bu):
    return pl.pallas_call(
        functools.partial(_node_conv_body, float(N)),
        grid=(N // RB_N,),
        in_specs=[
            pl.BlockSpec((RB_N, H), lambda i: (i, 0)),
            pl.BlockSpec((RB_N, H), lambda i: (i, 0)),
            pl.BlockSpec((RB_N, H), lambda i: (i, 0)),
            pl.BlockSpec((RB_N, H), lambda i: (i, 0)),
            pl.BlockSpec((H, H), lambda i: (0, 0)),
            pl.BlockSpec((H, C), lambda i: (0, 0)),
            pl.BlockSpec((1, C), lambda i: (0, 0)),
        ],
        out_specs=[
            pl.BlockSpec((RB_N, C), lambda i: (i, 0)),
            pl.BlockSpec((2, C), lambda i: (0, 0)),
        ],
        out_shape=[
            jax.ShapeDtypeStruct((N, C), jnp.float32),
            jax.ShapeDtypeStruct((2, C), jnp.float32),
        ],
        scratch_shapes=[pltpu.VMEM((2, C), jnp.float32)],
    )(p0, p1, h, db, cw, wu, bu.reshape(1, C))


def _node_out_body(cnt, u_ref, st_ref, g_ref, bb_ref, x_ref, o_ref):
    m, inv = _mean_inv(st_ref, cnt)
    z = (u_ref[...] - m) * inv * g_ref[...] + bb_ref[...] + x_ref[...]
    o_ref[...] = _silu(z)


def _node_out(u, st, g, bb, x):
    return pl.pallas_call(
        functools.partial(_node_out_body, float(N)),
        grid=(N // RB_N,),
        in_specs=[
            pl.BlockSpec((RB_N, C), lambda i: (i, 0)),
            pl.BlockSpec((2, C), lambda i: (0, 0)),
            pl.BlockSpec((1, C), lambda i: (0, 0)),
            pl.BlockSpec((1, C), lambda i: (0, 0)),
            pl.BlockSpec((RB_N, C), lambda i: (i, 0)),
        ],
        out_specs=pl.BlockSpec((RB_N, C), lambda i: (i, 0)),
        out_shape=jax.ShapeDtypeStruct((N, C), jnp.float32),
    )(u, st, g.reshape(1, C), bb.reshape(1, C), x)


def _sc_deg_body(idx_hbm, out_hbm, slab_v, ones_v, zrow_v, acc_sh):
    c = lax.axis_index("c")
    s = lax.axis_index("s")
    wid = c * 16 + s
    pltpu.sync_copy(idx_hbm.at[pl.ds(wid * 2 * NCH, 2 * NCH)], slab_v)
    for j in range(CH // 16):
        ones_v[pl.ds(j * 16, 16)] = jnp.ones((16,), jnp.float32)
    for j in range(ROWS_T // 16):
        zrow_v[pl.ds(j * 16, 16)] = jnp.zeros((16,), jnp.float32)
    pltpu.sync_copy(zrow_v, acc_sh.at[pl.ds(s * ROWS_T, ROWS_T)])
    plsc.subcore_barrier()

    def body(i, carry):
        pltpu.sync_copy(ones_v, acc_sh.at[slab_v.at[2 * i + 1]], add=True)
        return carry

    lax.fori_loop(0, NCH, body, 0)
    plsc.subcore_barrier()
    pltpu.sync_copy(acc_sh.at[pl.ds(s * ROWS_T, ROWS_T)],
                    out_hbm.at[c, pl.ds(s * ROWS_T, ROWS_T)])


@functools.cache
def _sc_deg():
    mesh = plsc.VectorSubcoreMesh(core_axis_name="c", subcore_axis_name="s")
    return pl.kernel(
        _sc_deg_body,
        out_type=jax.ShapeDtypeStruct((2, NPAD), jnp.float32),
        mesh=mesh,
        scratch_types=[
            pltpu.VMEM((2 * NCH, CH), jnp.int32),
            pltpu.VMEM((CH,), jnp.float32),
            pltpu.VMEM((ROWS_T,), jnp.float32),
            pltpu.VMEM_SHARED((NPAD,), jnp.float32),
        ],
        compiler_params=pltpu.CompilerParams(use_tc_tiling_on_sc=False),
    )


def _sc_agg_body(idx_hbm, hp_hbm, zeros_hbm, out_hbm,
                 slab_v, rows0_v, rows1_v, acc_sh, sem0, sem1):
    c = lax.axis_index("c")
    s = lax.axis_index("s")
    wid = c * 16 + s
    rows = (rows0_v, rows1_v)
    sems = (sem0, sem1)
    pltpu.sync_copy(idx_hbm.at[pl.ds(wid * 2 * NCH, 2 * NCH)], slab_v)
    pltpu.sync_copy(zeros_hbm.at[pl.ds(s * ROWS_T, ROWS_T)],
                    acc_sh.at[pl.ds(s * ROWS_T, ROWS_T)])
    plsc.subcore_barrier()

    for p in range(2):
        pltpu.async_copy(hp_hbm.at[slab_v.at[2 * p]], rows[p], sems[p])

    def body(k, carry):
        for p in range(2):
            i = 2 * k + p
            pltpu.make_async_copy(hp_hbm.at[slab_v.at[0]], rows[p],
                                  sems[p]).wait()
            pltpu.sync_copy(rows[p], acc_sh.at[slab_v.at[2 * i + 1]],
                            add=True)
            pltpu.async_copy(hp_hbm.at[slab_v.at[2 * (i + 2)]], rows[p],
                             sems[p])
        return carry

    lax.fori_loop(0, NCH // 2 - 1, body, 0)
    for p in range(2):
        i = NCH - 2 + p
        pltpu.make_async_copy(hp_hbm.at[slab_v.at[0]], rows[p],
                              sems[p]).wait()
        pltpu.sync_copy(rows[p], acc_sh.at[slab_v.at[2 * i + 1]], add=True)
    plsc.subcore_barrier()
    pltpu.sync_copy(acc_sh.at[pl.ds(s * ROWS_T, ROWS_T)],
                    out_hbm.at[c, pl.ds(s * ROWS_T, ROWS_T)])


@functools.cache
def _sc_agg():
    mesh = plsc.VectorSubcoreMesh(core_axis_name="c", subcore_axis_name="s")
    return pl.kernel(
        _sc_agg_body,
        out_type=jax.ShapeDtypeStruct((2, NPAD, H), jnp.float32),
        mesh=mesh,
        scratch_types=[
            pltpu.VMEM((2 * NCH, CH), jnp.int32),
            pltpu.VMEM((CH, H), jnp.float32),
            pltpu.VMEM((CH, H), jnp.float32),
            pltpu.VMEM_SHARED((NPAD, H), jnp.float32),
            pltpu.SemaphoreType.DMA,
            pltpu.SemaphoreType.DMA,
        ],
        compiler_params=pltpu.CompilerParams(use_tc_tiling_on_sc=False),
    )


def _interleave_idx(src, dst):
    s2 = src.reshape(NTILES * NCH, CH)
    d2 = dst.reshape(NTILES * NCH, CH)
    return jnp.stack([s2, d2], axis=1).reshape(NTILES * 2 * NCH, CH)


def _sc_deg_call(idx_il):
    return _sc_deg()(idx_il)


def _sc_agg_call(idx_il, hp, zeros):
    return _sc_agg()(idx_il, hp, zeros)


def kernel(x, edge_index, edge_attr, batch,
           w_dn_node, b_dn_node, w_dn_edge, b_dn_edge,
           bn1_node_w, bn1_node_b, bn1_edge_w, bn1_edge_b,
           conv_w,
           w_up_node, b_up_node, w_up_edge, b_up_edge,
           bn2_node_w, bn2_node_b, bn2_edge_w, bn2_edge_b):
    src = jnp.concatenate([edge_index[0], jnp.asarray(_PAD_SRC)])
    dst = jnp.concatenate([edge_index[1], jnp.asarray(_PAD_DST)])
    idx_il = _interleave_idx(src, dst)

    deg_p = _sc_deg_call(idx_il)
    d0 = deg_p[0].reshape(NPAD, 1)
    d1 = deg_p[1].reshape(NPAD, 1)

    pre_n, st1n = _proj_stats(x, w_dn_node, b_dn_node, N, RB_N)
    pre_e, st1e = _proj_stats(edge_attr, w_dn_edge, b_dn_edge, E, RB_E,
                              jnp.bfloat16)

    h_node, hp, db = _node_mid(pre_n, st1n, bn1_node_w, bn1_node_b, d0, d1)

    zeros = jnp.zeros((NPAD, H), jnp.float32)
    agg_p = _sc_agg_call(idx_il, hp, zeros)

    u_n, st2n = _node_conv(agg_p[0], agg_p[1], h_node, db,
                           conv_w, w_up_node, b_up_node)
    x_out = _node_out(u_n, st2n, bn2_node_w, bn2_node_b, x)

    h_edge, st2e = _bn_silu_proj(pre_e, st1e, bn1_edge_w, bn1_edge_b,
                                 w_up_edge, b_up_edge, E, RB_E)
    edge_out = _edge_out(h_edge, st2e, bn2_edge_w, bn2_edge_b,
                         w_up_edge, b_up_edge, edge_attr, E, RB_E)
    return x_out, edge_out


# speedup vs baseline: 9.7371x; 1.0599x over previous
import functools

import numpy as np
import jax
import jax.numpy as jnp
from jax import lax
from jax.experimental import pallas as pl
from jax.experimental.pallas import tpu as pltpu
from jax.experimental.pallas import tpu_sc as plsc

N = 10000
E = 160000
C = 256
H = C // 4
ALPHA = 0.3678
EPS = 1e-5

NTILES = 32
NPAD = 10240
ROWS_T = NPAD // 16
EPT = 5120
EPAD = NTILES * EPT
CH = 128
NCH = EPT // CH

RB_E = 8000
RB_N = 2000

_PAD_SRC = np.arange(EPAD - E, dtype=np.int32) % N
_PAD_DST = N + np.arange(EPAD - E, dtype=np.int32) % (NPAD - N)


def _silu(z):
    return z * (1.0 / (1.0 + jnp.exp(-z)))


def _mean_inv(st_ref, cnt):
    m = st_ref[0:1, :] / cnt
    var = st_ref[1:2, :] / cnt - m * m
    return m, lax.rsqrt(var + EPS)


def _proj_stats_body(x_ref, w_ref, b_ref, o_ref, st_ref, acc_ref):
    i = pl.program_id(0)
    nb = pl.num_programs(0)
    pre = jnp.dot(x_ref[...], w_ref[...], preferred_element_type=jnp.float32)
    pre = pre + b_ref[...]
    o_ref[...] = pre.astype(o_ref.dtype)

    @pl.when(i == 0)
    def _():
        acc_ref[...] = jnp.zeros_like(acc_ref)

    acc_ref[0:1, :] += jnp.sum(pre, axis=0, keepdims=True)
    acc_ref[1:2, :] += jnp.sum(pre * pre, axis=0, keepdims=True)

    @pl.when(i == nb - 1)
    def _():
        st_ref[...] = acc_ref[...]


def _proj_stats(x, w, b, rows, rb, out_dtype=jnp.float32):
    din, dout = w.shape
    return pl.pallas_call(
        _proj_stats_body,
        grid=(rows // rb,),
        in_specs=[
            pl.BlockSpec((rb, din), lambda i: (i, 0)),
            pl.BlockSpec((din, dout), lambda i: (0, 0)),
            pl.BlockSpec((1, dout), lambda i: (0, 0)),
        ],
        out_specs=[
            pl.BlockSpec((rb, dout), lambda i: (i, 0)),
            pl.BlockSpec((2, dout), lambda i: (0, 0)),
        ],
        out_shape=[
            jax.ShapeDtypeStruct((rows, dout), out_dtype),
            jax.ShapeDtypeStruct((2, dout), jnp.float32),
        ],
        scratch_shapes=[pltpu.VMEM((2, dout), jnp.float32)],
    )(x, w.reshape(din, dout), b.reshape(1, dout))


def _bn_silu_proj_body(cnt, pre_ref, st1_ref, g_ref, bb_ref, w_ref, b2_ref,
                       h_ref, st2_ref, acc_ref):
    i = pl.program_id(0)
    nb = pl.num_programs(0)
    m, inv = _mean_inv(st1_ref, cnt)
    z = (pre_ref[...].astype(jnp.float32) - m) * inv * g_ref[...] + bb_ref[...]
    h = _silu(z).astype(h_ref.dtype)
    h_ref[...] = h
    u = jnp.dot(h.astype(jnp.float32), w_ref[...],
                preferred_element_type=jnp.float32) + b2_ref[...]

    @pl.when(i == 0)
    def _():
        acc_ref[...] = jnp.zeros_like(acc_ref)

    acc_ref[0:1, :] += jnp.sum(u, axis=0, keepdims=True)
    acc_ref[1:2, :] += jnp.sum(u * u, axis=0, keepdims=True)

    @pl.when(i == nb - 1)
    def _():
        st2_ref[...] = acc_ref[...]


def _bn_silu_proj(pre, st1, g, bb, w, b2, rows, rb):
    din, dout = w.shape
    return pl.pallas_call(
        functools.partial(_bn_silu_proj_body, float(rows)),
        grid=(rows // rb,),
        in_specs=[
            pl.BlockSpec((rb, din), lambda i: (i, 0)),
            pl.BlockSpec((2, din), lambda i: (0, 0)),
            pl.BlockSpec((1, din), lambda i: (0, 0)),
            pl.BlockSpec((1, din), lambda i: (0, 0)),
            pl.BlockSpec((din, dout), lambda i: (0, 0)),
            pl.BlockSpec((1, dout), lambda i: (0, 0)),
        ],
        out_specs=[
            pl.BlockSpec((rb, din), lambda i: (i, 0)),
            pl.BlockSpec((2, dout), lambda i: (0, 0)),
        ],
        out_shape=[
            jax.ShapeDtypeStruct((rows, din), jnp.bfloat16),
            jax.ShapeDtypeStruct((2, dout), jnp.float32),
        ],
        scratch_shapes=[pltpu.VMEM((2, dout), jnp.float32)],
    )(pre, st1, g.reshape(1, din), bb.reshape(1, din), w, b2.reshape(1, dout))


def _edge_out_body(cnt, h_ref, st2_ref, g_ref, bb_ref, w_ref, b2_ref, res_ref,
                   o_ref):
    m, inv = _mean_inv(st2_ref, cnt)
    u = jnp.dot(h_ref[...].astype(jnp.float32), w_ref[...],
                preferred_element_type=jnp.float32)
    u = u + b2_ref[...]
    z = (u - m) * inv * g_ref[...] + bb_ref[...] + res_ref[...]
    o_ref[...] = _silu(z)


def _edge_out(h, st2, g, bb, w, b2, res, rows, rb):
    din, dout = w.shape
    return pl.pallas_call(
        functools.partial(_edge_out_body, float(rows)),
        grid=(rows // rb,),
        in_specs=[
            pl.BlockSpec((rb, din), lambda i: (i, 0)),
            pl.BlockSpec((2, dout), lambda i: (0, 0)),
            pl.BlockSpec((1, dout), lambda i: (0, 0)),
            pl.BlockSpec((1, dout), lambda i: (0, 0)),
            pl.BlockSpec((din, dout), lambda i: (0, 0)),
            pl.BlockSpec((1, dout), lambda i: (0, 0)),
            pl.BlockSpec((rb, dout), lambda i: (i, 0)),
        ],
        out_specs=pl.BlockSpec((rb, dout), lambda i: (i, 0)),
        out_shape=jax.ShapeDtypeStruct((rows, dout), jnp.float32),
    )(h, st2, g.reshape(1, dout), bb.reshape(1, dout), w, b2.reshape(1, dout),
      res)


def _node_mid_body(cnt, pre_ref, st1_ref, g_ref, bb_ref, d0_ref, d1_ref,
                   h_ref, hp_ref, db_ref):
    m, inv = _mean_inv(st1_ref, cnt)
    z = (pre_ref[...] - m) * inv * g_ref[...] + bb_ref[...]
    h = _silu(z)
    deg = d0_ref[...] + d1_ref[...] + 1.0
    dis = lax.rsqrt(deg)
    h_ref[...] = h
    hp_ref[...] = (h * dis).astype(hp_ref.dtype)
    db_ref[...] = jnp.broadcast_to(dis, h.shape)


def _node_mid(pre, st1, g, bb, d0, d1):
    return pl.pallas_call(
        functools.partial(_node_mid_body, float(N)),
        grid=(N // RB_N,),
        in_specs=[
            pl.BlockSpec((RB_N, H), lambda i: (i, 0)),
            pl.BlockSpec((2, H), lambda i: (0, 0)),
            pl.BlockSpec((1, H), lambda i: (0, 0)),
            pl.BlockSpec((1, H), lambda i: (0, 0)),
            pl.BlockSpec((RB_N, 1), lambda i: (i, 0)),
            pl.BlockSpec((RB_N, 1), lambda i: (i, 0)),
        ],
        out_specs=[
            pl.BlockSpec((RB_N, H), lambda i: (i, 0)),
            pl.BlockSpec((RB_N, H), lambda i: (i, 0)),
            pl.BlockSpec((RB_N, H), lambda i: (i, 0)),
        ],
        out_shape=[
            jax.ShapeDtypeStruct((N, H), jnp.float32),
            jax.ShapeDtypeStruct((N, H), jnp.bfloat16),
            jax.ShapeDtypeStruct((N, H), jnp.float32),
        ],
    )(pre, st1, g.reshape(1, H), bb.reshape(1, H), d0, d1)


def _node_conv_body(cnt, p0_ref, p1_ref, h_ref, db_ref, cw_ref, wu_ref,
                    bu_ref, u_ref, st_ref, acc_ref):
    i = pl.program_id(0)
    nb = pl.num_programs(0)
    db = db_ref[...]
    h = h_ref[...]
    psum = p0_ref[...].astype(jnp.float32) + p1_ref[...].astype(jnp.float32)
    agg = db * psum + db * db * h
    support = (1.0 - ALPHA) * agg + ALPHA * h
    conv = jnp.dot(support, cw_ref[...], preferred_element_type=jnp.float32)
    u = jnp.dot(conv, wu_ref[...], preferred_element_type=jnp.float32)
    u = u + bu_ref[...]
    u_ref[...] = u

    @pl.when(i == 0)
    def _():
        acc_ref[...] = jnp.zeros_like(acc_ref)

    acc_ref[0:1, :] += jnp.sum(u, axis=0, keepdims=True)
    acc_ref[1:2, :] += jnp.sum(u * u, axis=0, keepdims=True)

    @pl.when(i == nb - 1)
    def _():
        st_ref[...] = acc_ref[...]


def _node_conv(p0, p1, h, db, cw, wu, bu):
    return pl.pallas_call(
        functools.partial(_node_conv_body, float(N)),
        grid=(N // RB_N,),
        in_specs=[
            pl.BlockSpec((RB_N, H), lambda i: (i, 0)),
            pl.BlockSpec((RB_N, H), lambda i: (i, 0)),
            pl.BlockSpec((RB_N, H), lambda i: (i, 0)),
            pl.BlockSpec((RB_N, H), lambda i: (i, 0)),
            pl.BlockSpec((H, H), lambda i: (0, 0)),
            pl.BlockSpec((H, C), lambda i: (0, 0)),
            pl.BlockSpec((1, C), lambda i: (0, 0)),
        ],
        out_specs=[
            pl.BlockSpec((RB_N, C), lambda i: (i, 0)),
            pl.BlockSpec((2, C), lambda i: (0, 0)),
        ],
        out_shape=[
            jax.ShapeDtypeStruct((N, C), jnp.float32),
            jax.ShapeDtypeStruct((2, C), jnp.float32),
        ],
        scratch_shapes=[pltpu.VMEM((2, C), jnp.float32)],
    )(p0, p1, h, db, cw, wu, bu.reshape(1, C))


def _node_out_body(cnt, u_ref, st_ref, g_ref, bb_ref, x_ref, o_ref):
    m, inv = _mean_inv(st_ref, cnt)
    z = (u_ref[...] - m) * inv * g_ref[...] + bb_ref[...] + x_ref[...]
    o_ref[...] = _silu(z)


def _node_out(u, st, g, bb, x):
    return pl.pallas_call(
        functools.partial(_node_out_body, float(N)),
        grid=(N // RB_N,),
        in_specs=[
            pl.BlockSpec((RB_N, C), lambda i: (i, 0)),
            pl.BlockSpec((2, C), lambda i: (0, 0)),
            pl.BlockSpec((1, C), lambda i: (0, 0)),
            pl.BlockSpec((1, C), lambda i: (0, 0)),
            pl.BlockSpec((RB_N, C), lambda i: (i, 0)),
        ],
        out_specs=pl.BlockSpec((RB_N, C), lambda i: (i, 0)),
        out_shape=jax.ShapeDtypeStruct((N, C), jnp.float32),
    )(u, st, g.reshape(1, C), bb.reshape(1, C), x)


def _sc_deg_body(idx_hbm, out_hbm, slab_v, ones_v, zrow_v, acc_sh):
    c = lax.axis_index("c")
    s = lax.axis_index("s")
    wid = c * 16 + s
    pltpu.sync_copy(idx_hbm.at[pl.ds(wid * 2 * NCH, 2 * NCH)], slab_v)
    for j in range(CH // 16):
        ones_v[pl.ds(j * 16, 16)] = jnp.ones((16,), jnp.float32)
    for j in range(ROWS_T // 16):
        zrow_v[pl.ds(j * 16, 16)] = jnp.zeros((16,), jnp.float32)
    pltpu.sync_copy(zrow_v, acc_sh.at[pl.ds(s * ROWS_T, ROWS_T)])
    plsc.subcore_barrier()

    def body(i, carry):
        pltpu.sync_copy(ones_v, acc_sh.at[slab_v.at[2 * i + 1]], add=True)
        return carry

    lax.fori_loop(0, NCH, body, 0)
    plsc.subcore_barrier()
    pltpu.sync_copy(acc_sh.at[pl.ds(s * ROWS_T, ROWS_T)],
                    out_hbm.at[c, pl.ds(s * ROWS_T, ROWS_T)])


@functools.cache
def _sc_deg():
    mesh = plsc.VectorSubcoreMesh(core_axis_name="c", subcore_axis_name="s")
    return pl.kernel(
        _sc_deg_body,
        out_type=jax.ShapeDtypeStruct((2, NPAD), jnp.float32),
        mesh=mesh,
        scratch_types=[
            pltpu.VMEM((2 * NCH, CH), jnp.int32),
            pltpu.VMEM((CH,), jnp.float32),
            pltpu.VMEM((ROWS_T,), jnp.float32),
            pltpu.VMEM_SHARED((NPAD,), jnp.float32),
        ],
        compiler_params=pltpu.CompilerParams(use_tc_tiling_on_sc=False),
    )


def _sc_agg_body(idx_hbm, hp_hbm, zeros_hbm, out_hbm,
                 slab_v, rows0_v, rows1_v, acc_sh, sem0, sem1):
    c = lax.axis_index("c")
    s = lax.axis_index("s")
    wid = c * 16 + s
    rows = (rows0_v, rows1_v)
    sems = (sem0, sem1)
    pltpu.sync_copy(idx_hbm.at[pl.ds(wid * 2 * NCH, 2 * NCH)], slab_v)
    pltpu.sync_copy(zeros_hbm.at[pl.ds(s * ROWS_T, ROWS_T)],
                    acc_sh.at[pl.ds(s * ROWS_T, ROWS_T)])
    plsc.subcore_barrier()

    for p in range(2):
        pltpu.async_copy(hp_hbm.at[slab_v.at[2 * p]], rows[p], sems[p])

    def body(k, carry):
        for p in range(2):
            i = 2 * k + p
            pltpu.make_async_copy(hp_hbm.at[slab_v.at[0]], rows[p],
                                  sems[p]).wait()
            pltpu.sync_copy(rows[p], acc_sh.at[slab_v.at[2 * i + 1]],
                            add=True)
            pltpu.async_copy(hp_hbm.at[slab_v.at[2 * (i + 2)]], rows[p],
                             sems[p])
        return carry

    lax.fori_loop(0, NCH // 2 - 1, body, 0)
    for p in range(2):
        i = NCH - 2 + p
        pltpu.make_async_copy(hp_hbm.at[slab_v.at[0]], rows[p],
                              sems[p]).wait()
        pltpu.sync_copy(rows[p], acc_sh.at[slab_v.at[2 * i + 1]], add=True)
    plsc.subcore_barrier()
    pltpu.sync_copy(acc_sh.at[pl.ds(s * ROWS_T, ROWS_T)],
                    out_hbm.at[c, pl.ds(s * ROWS_T, ROWS_T)])


@functools.cache
def _sc_agg():
    mesh = plsc.VectorSubcoreMesh(core_axis_name="c", subcore_axis_name="s")
    return pl.kernel(
        _sc_agg_body,
        out_type=jax.ShapeDtypeStruct((2, NPAD, H), jnp.bfloat16),
        mesh=mesh,
        scratch_types=[
            pltpu.VMEM((2 * NCH, CH), jnp.int32),
            pltpu.VMEM((CH, H), jnp.bfloat16),
            pltpu.VMEM((CH, H), jnp.bfloat16),
            pltpu.VMEM_SHARED((NPAD, H), jnp.bfloat16),
            pltpu.SemaphoreType.DMA,
            pltpu.SemaphoreType.DMA,
        ],
        compiler_params=pltpu.CompilerParams(use_tc_tiling_on_sc=False),
    )


def _interleave_idx(src, dst):
    s2 = src.reshape(NTILES * NCH, CH)
    d2 = dst.reshape(NTILES * NCH, CH)
    return jnp.stack([s2, d2], axis=1).reshape(NTILES * 2 * NCH, CH)


def _sc_deg_call(idx_il):
    return _sc_deg()(idx_il)


def _sc_agg_call(idx_il, hp, zeros):
    return _sc_agg()(idx_il, hp, zeros)


def kernel(x, edge_index, edge_attr, batch,
           w_dn_node, b_dn_node, w_dn_edge, b_dn_edge,
           bn1_node_w, bn1_node_b, bn1_edge_w, bn1_edge_b,
           conv_w,
           w_up_node, b_up_node, w_up_edge, b_up_edge,
           bn2_node_w, bn2_node_b, bn2_edge_w, bn2_edge_b):
    src = jnp.concatenate([edge_index[0], jnp.asarray(_PAD_SRC)])
    dst = jnp.concatenate([edge_index[1], jnp.asarray(_PAD_DST)])
    idx_il = _interleave_idx(src, dst)

    deg_p = _sc_deg_call(idx_il)
    d0 = deg_p[0].reshape(NPAD, 1)
    d1 = deg_p[1].reshape(NPAD, 1)

    pre_n, st1n = _proj_stats(x, w_dn_node, b_dn_node, N, RB_N)

    h_node, hp, db = _node_mid(pre_n, st1n, bn1_node_w, bn1_node_b, d0, d1)

    zeros = jnp.zeros((NPAD, H), jnp.bfloat16)
    agg_p = _sc_agg_call(idx_il, hp, zeros)

    pre_e, st1e = _proj_stats(edge_attr, w_dn_edge, b_dn_edge, E, RB_E,
                              jnp.bfloat16)
    h_edge, st2e = _bn_silu_proj(pre_e, st1e, bn1_edge_w, bn1_edge_b,
                                 w_up_edge, b_up_edge, E, RB_E)
    edge_out = _edge_out(h_edge, st2e, bn2_edge_w, bn2_edge_b,
                         w_up_edge, b_up_edge, edge_attr, E, RB_E)

    u_n, st2n = _node_conv(agg_p[0], agg_p[1], h_node, db,
                           conv_w, w_up_node, b_up_node)
    x_out = _node_out(u_n, st2n, bn2_node_w, bn2_node_b, x)
    return x_out, edge_out
